# baseline (speedup 1.0000x reference)
import functools
import jax
import jax.numpy as jnp
from jax import lax
from jax.experimental import pallas as pl
from jax.experimental.pallas import tpu as pltpu
from jax.experimental.pallas import tpu_sc as plsc

_IOU_THR = 0.5
_CONF_THR = 0.05
_MAX_DET = 100
_N_PAD = 1024
_T_PAD = 128

f32 = jnp.float32
i32 = jnp.int32


def _softmax_body(conf_ref, box_ref, s_out, area_out, pos_out):
    B, C, NP = conf_ref.shape
    N = 1000
    z = conf_ref[...]
    zmax = jnp.max(z, axis=1, keepdims=True)
    e = jnp.exp(z - zmax)
    se = jnp.sum(e, axis=1, keepdims=True)
    s = e / se
    n_io3 = lax.broadcasted_iota(i32, (B, C, NP), 2)
    s = jnp.where(n_io3 < N, s, f32(-jnp.inf))
    s_out[...] = s
    y1 = box_ref[:, 0, :]
    x1 = box_ref[:, 1, :]
    y2 = box_ref[:, 2, :]
    x2 = box_ref[:, 3, :]
    area_out[...] = (jnp.maximum(y2 - y1, 0.0) *
                     jnp.maximum(x2 - x1, 0.0)).reshape(B, 1, NP)

    mask = (s > _CONF_THR).astype(f32).reshape(B * C, NP)
    r_io = lax.broadcasted_iota(i32, (NP, NP), 0)
    c_io = lax.broadcasted_iota(i32, (NP, NP), 1)
    tri = (r_io <= c_io).astype(f32)
    pref = jax.lax.dot_general(
        mask, tri, (((1,), (0,)), ((), ())),
        preferred_element_type=f32)
    pos_out[...] = pref.astype(i32).reshape(B, C, NP)


def _sc_nms_body(scores_h, pos_h, y1_h, x1_h, y2_h, x2_h, area_h,
                 cs_h, ci_h,
                 s_v, p_v, y1_v, x1_v, y2_v, x2_v, area_v,
                 sc_c, idx_c, y1c, x1c, y2c, x2c, areac, csv, civ):
    C = 80
    TASKS_PER = C // 4
    NV = _N_PAD // 16
    NEG = f32(-jnp.inf)
    lane = lax.iota(i32, 16)
    zero16i = jnp.zeros((16,), i32)

    wid = lax.axis_index("s") * 2 + lax.axis_index("c")
    b = wid // 4
    c0 = (wid % 4) * TASKS_PER

    pltpu.sync_copy(y1_h.at[b], y1_v)
    pltpu.sync_copy(x1_h.at[b], x1_v)
    pltpu.sync_copy(y2_h.at[b], y2_v)
    pltpu.sync_copy(x2_h.at[b], x2_v)
    pltpu.sync_copy(area_h.at[b], area_v)

    def task_body(ti, _):
        t = b * C + c0 + ti
        pltpu.sync_copy(scores_h.at[t], s_v)
        pltpu.sync_copy(pos_h.at[t], p_v)
        for j in range(_T_PAD // 16):
            csv[pl.ds(j * 16, 16)] = jnp.full((16,), -1.0, f32)
            civ[pl.ds(j * 16, 16)] = zero16i

        def compact(i, offv):
            sl = pl.ds(i * 16, 16)
            sv = s_v[sl]
            msk = sv > _CONF_THR
            pidx = p_v[sl] - 1
            plsc.store_scatter(sc_c, [pidx], sv, mask=msk)
            nidx = jnp.full((16,), i * 16, i32) + lane
            plsc.store_scatter(idx_c, [pidx], nidx, mask=msk)
            return offv + plsc.all_reduce_population_count(msk)

        offv = lax.fori_loop(0, NV, compact, zero16i)
        K = jnp.max(offv)
        nv = (K + 15) // 16
        Kv = jnp.full((16,), K, i32)

        def gatherc(j, _):
            sl = pl.ds(j * 16, 16)
            posv = jnp.full((16,), j * 16, i32) + lane
            idxv = jnp.where(posv < Kv, idx_c[sl], 0)
            idx_c[sl] = idxv
            y1c[sl] = plsc.load_gather(y1_v, [idxv])
            x1c[sl] = plsc.load_gather(x1_v, [idxv])
            y2c[sl] = plsc.load_gather(y2_v, [idxv])
            x2c[sl] = plsc.load_gather(x2_v, [idxv])
            areac[sl] = plsc.load_gather(area_v, [idxv])
            sc_c[sl] = jnp.where(posv < Kv, sc_c[sl], NEG)
            return 0

        lax.fori_loop(0, nv, gatherc, 0)

        def amx(j, carry):
            m, pb = carry
            v = sc_c[pl.ds(j * 16, 16)]
            posv = jnp.full((16,), j * 16, i32) + lane
            upd = v > m
            return jnp.where(upd, v, m), jnp.where(upd, posv, pb)

        m, pb = lax.fori_loop(0, nv, amx,
                              (jnp.full((16,), NEG, f32), zero16i))
        M = jnp.max(m)
        pstar = jnp.min(jnp.where(m == M, pb, 1 << 20))

        def g_cond(carry):
            tk, Mc, _ = carry
            return (Mc > _CONF_THR) & (tk < _MAX_DET)

        def g_body(carry):
            tk, Mc, p = carry
            tkv = jnp.full((16,), tk, i32)
            lane0 = lane == 0
            psplat = jnp.full((16,), p, i32)
            orig = plsc.load_gather(idx_c, [psplat])
            plsc.store_scatter(csv, [tkv], jnp.full((16,), Mc, f32),
                               mask=lane0)
            plsc.store_scatter(civ, [tkv], orig, mask=lane0)
            sy1 = plsc.load_gather(y1c, [psplat])
            sx1 = plsc.load_gather(x1c, [psplat])
            sy2 = plsc.load_gather(y2c, [psplat])
            sx2 = plsc.load_gather(x2c, [psplat])
            asel = plsc.load_gather(areac, [psplat])

            def sup(j, carry2):
                m2, pb2 = carry2
                sl = pl.ds(j * 16, 16)
                sv = sc_c[sl]
                iy1 = jnp.maximum(sy1, y1c[sl])
                ix1 = jnp.maximum(sx1, x1c[sl])
                iy2 = jnp.minimum(sy2, y2c[sl])
                ix2 = jnp.minimum(sx2, x2c[sl])
                inter = (jnp.maximum(iy2 - iy1, 0.0) *
                         jnp.maximum(ix2 - ix1, 0.0))
                union = asel + areac[sl] - inter
                iou = inter / (union + 1e-8)
                posv = jnp.full((16,), j * 16, i32) + lane
                kill = (iou > _IOU_THR) | (posv == psplat)
                sv2 = jnp.where(kill, NEG, sv)
                sc_c[sl] = sv2
                upd = sv2 > m2
                return jnp.where(upd, sv2, m2), jnp.where(upd, posv, pb2)

            m2, pb2 = lax.fori_loop(0, nv, sup,
                                    (jnp.full((16,), NEG, f32), zero16i))
            M2 = jnp.max(m2)
            p2 = jnp.min(jnp.where(m2 == M2, pb2, 1 << 20))
            return tk + 1, M2, p2

        lax.while_loop(g_cond, g_body, (jnp.array(0, i32), M, pstar))
        pltpu.sync_copy(csv, cs_h.at[t])
        pltpu.sync_copy(civ, ci_h.at[t])
        return 0

    lax.fori_loop(0, TASKS_PER, task_body, 0)


def _run_sc_stage(scores2d, pos2d, y1, x1, y2, x2, area):
    BC, NP = scores2d.shape
    sc_nms = functools.partial(
        pl.kernel,
        mesh=plsc.VectorSubcoreMesh(core_axis_name="c", subcore_axis_name="s"),
        compiler_params=pltpu.CompilerParams(needs_layout_passes=False),
        out_type=[
            jax.ShapeDtypeStruct((BC, _T_PAD), f32),
            jax.ShapeDtypeStruct((BC, _T_PAD), i32),
        ],
        scratch_types=[
            pltpu.VMEM((NP,), f32),
            pltpu.VMEM((NP,), i32),
            pltpu.VMEM((NP,), f32),
            pltpu.VMEM((NP,), f32),
            pltpu.VMEM((NP,), f32),
            pltpu.VMEM((NP,), f32),
            pltpu.VMEM((NP,), f32),
            pltpu.VMEM((NP,), f32),
            pltpu.VMEM((NP,), i32),
            pltpu.VMEM((NP,), f32),
            pltpu.VMEM((NP,), f32),
            pltpu.VMEM((NP,), f32),
            pltpu.VMEM((NP,), f32),
            pltpu.VMEM((NP,), f32),
            pltpu.VMEM((_T_PAD,), f32),
            pltpu.VMEM((_T_PAD,), i32),
        ],
    )(_sc_nms_body)
    return sc_nms(scores2d, pos2d, y1, x1, y2, x2, area)


def _topk_body(cs_in, ci_in, box_ref, conf_o, cls_o, box_o, num_o, cs_ref):
    B, C, TP = cs_in.shape
    NP = box_ref.shape[2]
    T = _MAX_DET
    NEG = f32(-jnp.inf)

    conf_o[...] = jnp.zeros(conf_o.shape, f32)
    cls_o[...] = jnp.zeros(cls_o.shape, f32)
    box_o[...] = jnp.zeros(box_o.shape, f32)
    cs_ref[...] = cs_in[...]

    t_io3 = lax.broadcasted_iota(i32, (B, C, TP), 2)
    c_io3 = lax.broadcasted_iota(i32, (B, C, TP), 1)
    c_io2 = lax.broadcasted_iota(i32, (B, C), 1)
    n_io2 = lax.broadcasted_iota(i32, (B, NP), 1)

    def topk_cond(c):
        k, go = c
        return go & (k < T)

    def topk_step(c):
        k, _ = c
        cs = cs_ref[...]
        m_t = jnp.max(cs, axis=2)
        tstar = jnp.min(jnp.where(cs == m_t[:, :, None], t_io3, TP),
                        axis=2)
        m_b = jnp.max(m_t, axis=1)
        cstar = jnp.min(jnp.where(m_t == m_b[:, None], c_io2, C), axis=1)
        tsel = jnp.min(jnp.where(c_io2 == cstar[:, None], tstar, 10000),
                       axis=1)
        oh3 = ((t_io3 == tsel[:, None, None]) &
               (c_io3 == cstar[:, None, None]))
        bidx = jnp.sum(jnp.sum(jnp.where(oh3, ci_in[...], 0), axis=2),
                       axis=1)
        cs_ref[...] = jnp.where(oh3, NEG, cs)
        valid = m_b > 0.0
        conf_o[0, k] = jnp.where(valid, m_b, 0.0).reshape(1, B)
        cls_o[0, k] = jnp.where(valid, cstar.astype(f32), 0.0).reshape(1, B)
        ohn = (n_io2 == bidx[:, None]) & valid[:, None]
        rows = [
            jnp.sum(jnp.where(ohn, box_ref[:, j, :], 0.0),
                    axis=1).reshape(1, B)
            for j in range(4)
        ]
        box_o[0, k] = jnp.concatenate(rows, axis=0)
        return k + 1, jnp.any(valid)

    lax.while_loop(topk_cond, topk_step,
                   (jnp.array(0, i32), jnp.array(True)))
    num_o[...] = jnp.sum((conf_o[0] > 0.0).astype(i32),
                         axis=0).reshape(1, 1, B)


def kernel(box_pred, confidence_pred):
    B, N, C = confidence_pred.shape
    NP = _N_PAD
    T = _MAX_DET
    conf_t = jnp.pad(jnp.transpose(confidence_pred, (0, 2, 1)),
                     ((0, 0), (0, 0), (0, NP - N)))
    box_t = jnp.pad(jnp.transpose(box_pred, (0, 2, 1)),
                    ((0, 0), (0, 0), (0, NP - N)))

    Bh = B // 2
    scores, area, pos = pl.pallas_call(
        _softmax_body,
        grid=(2,),
        in_specs=[
            pl.BlockSpec((Bh, C, NP), lambda i: (i, 0, 0)),
            pl.BlockSpec((Bh, 4, NP), lambda i: (i, 0, 0)),
        ],
        out_specs=[
            pl.BlockSpec((Bh, C, NP), lambda i: (i, 0, 0)),
            pl.BlockSpec((Bh, 1, NP), lambda i: (i, 0, 0)),
            pl.BlockSpec((Bh, C, NP), lambda i: (i, 0, 0)),
        ],
        out_shape=[
            jax.ShapeDtypeStruct((B, C, NP), f32),
            jax.ShapeDtypeStruct((B, 1, NP), f32),
            jax.ShapeDtypeStruct((B, C, NP), i32),
        ],
        compiler_params=pltpu.CompilerParams(
            dimension_semantics=("parallel",)),
    )(conf_t, box_t)

    area = area.reshape(B, NP)
    cand_s, cand_i = _run_sc_stage(
        scores.reshape(B * C, NP), pos.reshape(B * C, NP),
        box_t[:, 0, :], box_t[:, 1, :], box_t[:, 2, :], box_t[:, 3, :],
        area)

    conf_o, cls_o, box_o, num_o = pl.pallas_call(
        _topk_body,
        grid=(2,),
        in_specs=[
            pl.BlockSpec((Bh, C, _T_PAD), lambda i: (i, 0, 0)),
            pl.BlockSpec((Bh, C, _T_PAD), lambda i: (i, 0, 0)),
            pl.BlockSpec((Bh, 4, NP), lambda i: (i, 0, 0)),
        ],
        out_specs=[
            pl.BlockSpec((1, T, 1, Bh), lambda i: (i, 0, 0, 0)),
            pl.BlockSpec((1, T, 1, Bh), lambda i: (i, 0, 0, 0)),
            pl.BlockSpec((1, T, 4, Bh), lambda i: (i, 0, 0, 0)),
            pl.BlockSpec((1, 1, Bh), lambda i: (i, 0, 0)),
        ],
        out_shape=[
            jax.ShapeDtypeStruct((2, T, 1, Bh), f32),
            jax.ShapeDtypeStruct((2, T, 1, Bh), f32),
            jax.ShapeDtypeStruct((2, T, 4, Bh), f32),
            jax.ShapeDtypeStruct((2, 1, Bh), jnp.int32),
        ],
        scratch_shapes=[
            pltpu.VMEM((Bh, C, _T_PAD), f32),
        ],
        compiler_params=pltpu.CompilerParams(
            dimension_semantics=("parallel",)),
    )(cand_s.reshape(B, C, _T_PAD), cand_i.reshape(B, C, _T_PAD), box_t)

    boxes_out = jnp.transpose(box_o, (0, 3, 1, 2)).reshape(B, T, 4)
    conf_out = jnp.transpose(conf_o[:, :, 0, :], (0, 2, 1)).reshape(B, T)
    cls_out = jnp.transpose(cls_o[:, :, 0, :], (0, 2, 1)).reshape(B, T)
    num = num_o.reshape(B)
    return boxes_out, conf_out, cls_out, num


# speedup vs baseline: 1.2497x; 1.2497x over previous
import functools
import jax
import jax.numpy as jnp
from jax import lax
from jax.experimental import pallas as pl
from jax.experimental.pallas import tpu as pltpu
from jax.experimental.pallas import tpu_sc as plsc

_IOU_THR = 0.5
_CONF_THR = 0.05
_MAX_DET = 100
_N_PAD = 1024
_NC_PAD = _N_PAD + 16
_T_PAD = 128

f32 = jnp.float32
i32 = jnp.int32


def _softmax_body(conf_ref, box_ref, s_out, area_out):
    B, C, NP = conf_ref.shape
    N = 1000
    z = conf_ref[...]
    zmax = jnp.max(z, axis=1, keepdims=True)
    e = jnp.exp(z - zmax)
    se = jnp.sum(e, axis=1, keepdims=True)
    s = e / se
    n_io3 = lax.broadcasted_iota(i32, (B, C, NP), 2)
    s_out[...] = jnp.where(n_io3 < N, s, f32(-jnp.inf))
    y1 = box_ref[:, 0, :]
    x1 = box_ref[:, 1, :]
    y2 = box_ref[:, 2, :]
    x2 = box_ref[:, 3, :]
    area_out[...] = jnp.maximum(y2 - y1, 0.0) * jnp.maximum(x2 - x1, 0.0)


def _sc_nms_body(scores_h, y1_h, x1_h, y2_h, x2_h, area_h,
                 cs_h, ci_h,
                 s_v, y1_v, x1_v, y2_v, x2_v, area_v,
                 sc_c, idx_c, y1c, x1c, y2c, x2c, areac, csv, civ,
                 sem_b, sem_s, sem_oc, sem_oi):
    C = 80
    TASKS_PER = C // 4
    NV = _N_PAD // 16
    NEG = f32(-jnp.inf)
    lane = lax.iota(i32, 16)
    zero16f = jnp.zeros((16,), f32)
    zero16i = jnp.zeros((16,), i32)

    wid = lax.axis_index("s") * 2 + lax.axis_index("c")
    b = wid // 4
    c0 = (wid % 4) * TASKS_PER

    cb = [pltpu.async_copy(y1_h.at[b], y1_v, sem_b),
          pltpu.async_copy(x1_h.at[b], x1_v, sem_b),
          pltpu.async_copy(y2_h.at[b], y2_v, sem_b),
          pltpu.async_copy(x2_h.at[b], x2_v, sem_b),
          pltpu.async_copy(area_h.at[b], area_v, sem_b)]
    for c in cb:
        c.wait()

    def task_body(ti, _):
        t = b * C + c0 + ti
        c_s = pltpu.async_copy(scores_h.at[t], s_v, sem_s)

        @pl.when(ti > 0)
        def _wait_prev_out():
            pltpu.make_async_copy(csv, cs_h.at[t], sem_oc).wait()
            pltpu.make_async_copy(civ, ci_h.at[t], sem_oi).wait()

        for j in range(_T_PAD // 16):
            csv[pl.ds(j * 16, 16)] = jnp.full((16,), -1.0, f32)
            civ[pl.ds(j * 16, 16)] = zero16i
        c_s.wait()

        def compact(i, offv):
            sv = s_v[pl.ds(i * 16, 16)]
            msk = sv > _CONF_THR
            cum = plsc.cumsum(msk.astype(i32))
            pos = offv + cum - 1
            plsc.store_scatter(sc_c, [pos], sv, mask=msk)
            nidx = jnp.full((16,), i * 16, i32) + lane
            plsc.store_scatter(idx_c, [pos], nidx, mask=msk)
            return offv + plsc.all_reduce_population_count(msk)

        offv = lax.fori_loop(0, NV, compact, zero16i)
        K = jnp.max(offv)
        nv = (K + 15) // 16
        Kv = jnp.full((16,), K, i32)

        def gatherc(j, _):
            sl = pl.ds(j * 16, 16)
            posv = jnp.full((16,), j * 16, i32) + lane
            idxv = jnp.where(posv < Kv, idx_c[sl], 0)
            idx_c[sl] = idxv
            y1c[sl] = plsc.load_gather(y1_v, [idxv])
            x1c[sl] = plsc.load_gather(x1_v, [idxv])
            y2c[sl] = plsc.load_gather(y2_v, [idxv])
            x2c[sl] = plsc.load_gather(x2_v, [idxv])
            areac[sl] = plsc.load_gather(area_v, [idxv])
            sc_c[sl] = jnp.where(posv < Kv, sc_c[sl], NEG)
            return 0

        lax.fori_loop(0, nv, gatherc, 0)

        def amx(j, carry):
            m, pb = carry
            v = sc_c[pl.ds(j * 16, 16)]
            posv = jnp.full((16,), j * 16, i32) + lane
            upd = v > m
            return jnp.where(upd, v, m), jnp.where(upd, posv, pb)

        m, pb = lax.fori_loop(0, nv, amx,
                              (jnp.full((16,), NEG, f32), zero16i))
        M = jnp.max(m)
        pstar = jnp.min(jnp.where(m == M, pb, 1 << 20))

        def g_cond(carry):
            tk, Mc, _ = carry
            return (Mc > _CONF_THR) & (tk < _MAX_DET)

        def g_body(carry):
            tk, Mc, p = carry
            tkv = jnp.full((16,), tk, i32)
            lane0 = lane == 0
            psplat = jnp.full((16,), p, i32)
            orig = plsc.load_gather(idx_c, [psplat])
            plsc.store_scatter(csv, [tkv], jnp.full((16,), Mc, f32),
                               mask=lane0)
            plsc.store_scatter(civ, [tkv], orig, mask=lane0)
            sy1 = plsc.load_gather(y1c, [psplat])
            sx1 = plsc.load_gather(x1c, [psplat])
            sy2 = plsc.load_gather(y2c, [psplat])
            sx2 = plsc.load_gather(x2c, [psplat])
            asel = plsc.load_gather(areac, [psplat])

            def sup(j, carry2):
                m2, pb2 = carry2
                sl = pl.ds(j * 16, 16)
                sv = sc_c[sl]
                iy1 = jnp.maximum(sy1, y1c[sl])
                ix1 = jnp.maximum(sx1, x1c[sl])
                iy2 = jnp.minimum(sy2, y2c[sl])
                ix2 = jnp.minimum(sx2, x2c[sl])
                inter = (jnp.maximum(iy2 - iy1, 0.0) *
                         jnp.maximum(ix2 - ix1, 0.0))
                union = asel + areac[sl] - inter
                iou = inter / (union + 1e-8)
                posv = jnp.full((16,), j * 16, i32) + lane
                kill = (iou > _IOU_THR) | (posv == psplat)
                sv2 = jnp.where(kill, NEG, sv)
                sc_c[sl] = sv2
                upd = sv2 > m2
                return jnp.where(upd, sv2, m2), jnp.where(upd, posv, pb2)

            m2, pb2 = lax.fori_loop(0, nv, sup,
                                    (jnp.full((16,), NEG, f32), zero16i))
            M2 = jnp.max(m2)
            p2 = jnp.min(jnp.where(m2 == M2, pb2, 1 << 20))
            return tk + 1, M2, p2

        lax.while_loop(g_cond, g_body, (jnp.array(0, i32), M, pstar))
        pltpu.async_copy(csv, cs_h.at[t], sem_oc)
        pltpu.async_copy(civ, ci_h.at[t], sem_oi)
        return 0

    lax.fori_loop(0, TASKS_PER, task_body, 0)
    t_last = b * C + c0 + TASKS_PER - 1
    pltpu.make_async_copy(csv, cs_h.at[t_last], sem_oc).wait()
    pltpu.make_async_copy(civ, ci_h.at[t_last], sem_oi).wait()


def _run_sc_stage(scores2d, y1, x1, y2, x2, area):
    BC, NP = scores2d.shape
    sc_nms = functools.partial(
        pl.kernel,
        mesh=plsc.VectorSubcoreMesh(core_axis_name="c", subcore_axis_name="s"),
        compiler_params=pltpu.CompilerParams(needs_layout_passes=False),
        out_type=[
            jax.ShapeDtypeStruct((BC, _T_PAD), f32),
            jax.ShapeDtypeStruct((BC, _T_PAD), i32),
        ],
        scratch_types=[
            pltpu.VMEM((NP,), f32),
            pltpu.VMEM((NP,), f32),
            pltpu.VMEM((NP,), f32),
            pltpu.VMEM((NP,), f32),
            pltpu.VMEM((NP,), f32),
            pltpu.VMEM((NP,), f32),
            pltpu.VMEM((_NC_PAD,), f32),
            pltpu.VMEM((_NC_PAD,), i32),
            pltpu.VMEM((_NC_PAD,), f32),
            pltpu.VMEM((_NC_PAD,), f32),
            pltpu.VMEM((_NC_PAD,), f32),
            pltpu.VMEM((_NC_PAD,), f32),
            pltpu.VMEM((_NC_PAD,), f32),
            pltpu.VMEM((_T_PAD,), f32),
            pltpu.VMEM((_T_PAD,), i32),
            pltpu.SemaphoreType.DMA,
            pltpu.SemaphoreType.DMA,
            pltpu.SemaphoreType.DMA,
            pltpu.SemaphoreType.DMA,
        ],
    )(_sc_nms_body)
    return sc_nms(scores2d, y1, x1, y2, x2, area)


def _topk_body(cs_in, ci_in, box_ref, conf_o, cls_o, box_o, num_o, cs_ref):
    B, C, TP = cs_in.shape
    NP = box_ref.shape[2]
    T = _MAX_DET
    NEG = f32(-jnp.inf)

    conf_o[...] = jnp.zeros(conf_o.shape, f32)
    cls_o[...] = jnp.zeros(cls_o.shape, f32)
    box_o[...] = jnp.zeros(box_o.shape, f32)
    cs_ref[...] = cs_in[...]

    t_io3 = lax.broadcasted_iota(i32, (B, C, TP), 2)
    c_io3 = lax.broadcasted_iota(i32, (B, C, TP), 1)
    c_io2 = lax.broadcasted_iota(i32, (B, C), 1)
    n_io2 = lax.broadcasted_iota(i32, (B, NP), 1)

    def topk_cond(c):
        k, go = c
        return go & (k < T)

    def topk_step(c):
        k, _ = c
        cs = cs_ref[...]
        m_t = jnp.max(cs, axis=2)
        tstar = jnp.min(jnp.where(cs == m_t[:, :, None], t_io3, TP),
                        axis=2)
        m_b = jnp.max(m_t, axis=1)
        cstar = jnp.min(jnp.where(m_t == m_b[:, None], c_io2, C), axis=1)
        tsel = jnp.min(jnp.where(c_io2 == cstar[:, None], tstar, 10000),
                       axis=1)
        oh3 = ((t_io3 == tsel[:, None, None]) &
               (c_io3 == cstar[:, None, None]))
        bidx = jnp.sum(jnp.sum(jnp.where(oh3, ci_in[...], 0), axis=2),
                       axis=1)
        cs_ref[...] = jnp.where(oh3, NEG, cs)
        valid = m_b > 0.0
        conf_o[k] = jnp.where(valid, m_b, 0.0).reshape(1, B)
        cls_o[k] = jnp.where(valid, cstar.astype(f32), 0.0).reshape(1, B)
        ohn = (n_io2 == bidx[:, None]) & valid[:, None]
        rows = [
            jnp.sum(jnp.where(ohn, box_ref[:, j, :], 0.0),
                    axis=1).reshape(1, B)
            for j in range(4)
        ]
        box_o[k] = jnp.concatenate(rows, axis=0)
        return k + 1, jnp.any(valid)

    lax.while_loop(topk_cond, topk_step,
                   (jnp.array(0, i32), jnp.array(True)))
    num_o[...] = jnp.sum((conf_o[...] > 0.0).astype(i32), axis=0)


def kernel(box_pred, confidence_pred):
    B, N, C = confidence_pred.shape
    NP = _N_PAD
    T = _MAX_DET
    conf_t = jnp.pad(jnp.transpose(confidence_pred, (0, 2, 1)),
                     ((0, 0), (0, 0), (0, NP - N)))
    box_t = jnp.pad(jnp.transpose(box_pred, (0, 2, 1)),
                    ((0, 0), (0, 0), (0, NP - N)))

    scores, area = pl.pallas_call(
        _softmax_body,
        out_shape=[
            jax.ShapeDtypeStruct((B, C, NP), f32),
            jax.ShapeDtypeStruct((B, NP), f32),
        ],
    )(conf_t, box_t)

    cand_s, cand_i = _run_sc_stage(
        scores.reshape(B * C, NP),
        box_t[:, 0, :], box_t[:, 1, :], box_t[:, 2, :], box_t[:, 3, :],
        area)

    conf_o, cls_o, box_o, num_o = pl.pallas_call(
        _topk_body,
        out_shape=[
            jax.ShapeDtypeStruct((T, 1, B), f32),
            jax.ShapeDtypeStruct((T, 1, B), f32),
            jax.ShapeDtypeStruct((T, 4, B), f32),
            jax.ShapeDtypeStruct((1, B), jnp.int32),
        ],
        scratch_shapes=[
            pltpu.VMEM((B, C, _T_PAD), f32),
        ],
    )(cand_s.reshape(B, C, _T_PAD), cand_i.reshape(B, C, _T_PAD), box_t)

    boxes_out = jnp.transpose(box_o, (2, 0, 1))
    conf_out = conf_o[:, 0, :].T
    cls_out = cls_o[:, 0, :].T
    num = num_o[0]
    return boxes_out, conf_out, cls_out, num


# speedup vs baseline: 2.6944x; 2.1560x over previous
import functools
import jax
import jax.numpy as jnp
from jax import lax
from jax.experimental import pallas as pl
from jax.experimental.pallas import tpu as pltpu
from jax.experimental.pallas import tpu_sc as plsc

_IOU_THR = 0.5
_CONF_THR = 0.05
_MAX_DET = 100
_N_PAD = 1024
_NC_PAD = _N_PAD + 16
_T_PAD = 128

f32 = jnp.float32
i32 = jnp.int32


def _softmax_body(conf_ref, box_ref, s_out, area_out):
    B, C, NP = conf_ref.shape
    N = 1000
    z = conf_ref[...]
    zmax = jnp.max(z, axis=1, keepdims=True)
    e = jnp.exp(z - zmax)
    se = jnp.sum(e, axis=1, keepdims=True)
    s = e / se
    n_io3 = lax.broadcasted_iota(i32, (B, C, NP), 2)
    s_out[...] = jnp.where(n_io3 < N, s, f32(-jnp.inf))
    y1 = box_ref[:, 0, :]
    x1 = box_ref[:, 1, :]
    y2 = box_ref[:, 2, :]
    x2 = box_ref[:, 3, :]
    area_out[...] = jnp.maximum(y2 - y1, 0.0) * jnp.maximum(x2 - x1, 0.0)


def _sc_nms_body(scores_h, y1_h, x1_h, y2_h, x2_h, area_h,
                 cs_h, ci_h, conf_h, cls_h, box_h, num_h,
                 s_v, y1_v, x1_v, y2_v, x2_v, area_v,
                 sc_c, idx_c, y1c, x1c, y2c, x2c, areac, csv, civ,
                 cs_m, ci_m, mo_conf, mo_cls,
                 mo_y1, mo_x1, mo_y2, mo_x2, mo_num,
                 sem_b, sem_s, sem_oc, sem_oi, sem_m):
    C = 80
    TASKS_PER = C // 4
    NV = _N_PAD // 16
    NEG = f32(-jnp.inf)
    lane = lax.iota(i32, 16)
    zero16i = jnp.zeros((16,), i32)

    cid = lax.axis_index("c")
    sid = lax.axis_index("s")
    wid = cid * 16 + sid
    b = wid // 4
    c0 = (wid % 4) * TASKS_PER

    cb = [pltpu.async_copy(y1_h.at[b], y1_v, sem_b),
          pltpu.async_copy(x1_h.at[b], x1_v, sem_b),
          pltpu.async_copy(y2_h.at[b], y2_v, sem_b),
          pltpu.async_copy(x2_h.at[b], x2_v, sem_b),
          pltpu.async_copy(area_h.at[b], area_v, sem_b)]
    for c in cb:
        c.wait()

    def task_body(ti, _):
        t = b * C + c0 + ti
        c_s = pltpu.async_copy(scores_h.at[t], s_v, sem_s)

        @pl.when(ti > 0)
        def _wait_prev_out():
            pltpu.make_async_copy(csv, cs_h.at[t], sem_oc).wait()
            pltpu.make_async_copy(civ, ci_h.at[t], sem_oi).wait()

        for j in range(_T_PAD // 16):
            csv[pl.ds(j * 16, 16)] = jnp.full((16,), -1.0, f32)
            civ[pl.ds(j * 16, 16)] = zero16i
        c_s.wait()

        def compact(i, offv):
            sl = pl.ds(i * 16, 16)
            sv = s_v[sl]
            msk = sv > _CONF_THR
            cum = plsc.cumsum(msk.astype(i32))
            pos = offv + cum - 1
            plsc.store_scatter(sc_c, [pos], sv, mask=msk)
            nidx = jnp.full((16,), i * 16, i32) + lane
            plsc.store_scatter(idx_c, [pos], nidx, mask=msk)
            return offv + plsc.all_reduce_population_count(msk)

        offv = lax.fori_loop(0, NV, compact, zero16i)
        K = jnp.max(offv)
        nv = (K + 15) // 16
        Kv = jnp.full((16,), K, i32)

        def gatherc(j, _):
            sl = pl.ds(j * 16, 16)
            posv = jnp.full((16,), j * 16, i32) + lane
            idxv = jnp.where(posv < Kv, idx_c[sl], 0)
            idx_c[sl] = idxv
            y1c[sl] = plsc.load_gather(y1_v, [idxv])
            x1c[sl] = plsc.load_gather(x1_v, [idxv])
            y2c[sl] = plsc.load_gather(y2_v, [idxv])
            x2c[sl] = plsc.load_gather(x2_v, [idxv])
            areac[sl] = plsc.load_gather(area_v, [idxv])
            sc_c[sl] = jnp.where(posv < Kv, sc_c[sl], NEG)
            return 0

        lax.fori_loop(0, nv, gatherc, 0)

        def amx(j, carry):
            m, pb = carry
            v = sc_c[pl.ds(j * 16, 16)]
            posv = jnp.full((16,), j * 16, i32) + lane
            upd = v > m
            return jnp.where(upd, v, m), jnp.where(upd, posv, pb)

        m, pb = lax.fori_loop(0, nv, amx,
                              (jnp.full((16,), NEG, f32), zero16i))
        M = jnp.max(m)
        pstar = jnp.min(jnp.where(m == M, pb, 1 << 20))

        def g_cond(carry):
            tk, Mc, _ = carry
            return (Mc > _CONF_THR) & (tk < _MAX_DET)

        def g_body(carry):
            tk, Mc, p = carry
            tkv = jnp.full((16,), tk, i32)
            lane0 = lane == 0
            psplat = jnp.full((16,), p, i32)
            orig = plsc.load_gather(idx_c, [psplat])
            plsc.store_scatter(csv, [tkv], jnp.full((16,), Mc, f32),
                               mask=lane0)
            plsc.store_scatter(civ, [tkv], orig, mask=lane0)
            sy1 = plsc.load_gather(y1c, [psplat])
            sx1 = plsc.load_gather(x1c, [psplat])
            sy2 = plsc.load_gather(y2c, [psplat])
            sx2 = plsc.load_gather(x2c, [psplat])
            asel = plsc.load_gather(areac, [psplat])

            def sup(j, carry2):
                m2, pb2 = carry2
                sl = pl.ds(j * 16, 16)
                sv = sc_c[sl]
                iy1 = jnp.maximum(sy1, y1c[sl])
                ix1 = jnp.maximum(sx1, x1c[sl])
                iy2 = jnp.minimum(sy2, y2c[sl])
                ix2 = jnp.minimum(sx2, x2c[sl])
                inter = (jnp.maximum(iy2 - iy1, 0.0) *
                         jnp.maximum(ix2 - ix1, 0.0))
                union = asel + areac[sl] - inter
                iou = inter / (union + 1e-8)
                posv = jnp.full((16,), j * 16, i32) + lane
                kill = (iou > _IOU_THR) | (posv == psplat)
                sv2 = jnp.where(kill, NEG, sv)
                sc_c[sl] = sv2
                upd = sv2 > m2
                return jnp.where(upd, sv2, m2), jnp.where(upd, posv, pb2)

            m2, pb2 = lax.fori_loop(0, nv, sup,
                                    (jnp.full((16,), NEG, f32), zero16i))
            M2 = jnp.max(m2)
            p2 = jnp.min(jnp.where(m2 == M2, pb2, 1 << 20))
            return tk + 1, M2, p2

        lax.while_loop(g_cond, g_body, (jnp.array(0, i32), M, pstar))
        pltpu.async_copy(csv, cs_h.at[t], sem_oc)
        pltpu.async_copy(civ, ci_h.at[t], sem_oi)
        return 0

    lax.fori_loop(0, TASKS_PER, task_body, 0)
    t_last = b * C + c0 + TASKS_PER - 1
    pltpu.make_async_copy(csv, cs_h.at[t_last], sem_oc).wait()
    pltpu.make_async_copy(civ, ci_h.at[t_last], sem_oi).wait()

    plsc.subcore_barrier()

    @pl.when(sid < 4)
    def _merge():
        bm = cid * 4 + sid
        cm = [pltpu.async_copy(cs_h.at[pl.ds(bm * C, C)], cs_m, sem_m),
              pltpu.async_copy(ci_h.at[pl.ds(bm * C, C)], ci_m, sem_m),
              pltpu.async_copy(y1_h.at[bm], y1_v, sem_m),
              pltpu.async_copy(x1_h.at[bm], x1_v, sem_m),
              pltpu.async_copy(y2_h.at[bm], y2_v, sem_m),
              pltpu.async_copy(x2_h.at[bm], x2_v, sem_m)]
        for c in cm:
            c.wait()
        for j in range(_T_PAD // 16):
            sl = pl.ds(j * 16, 16)
            mo_conf[sl] = jnp.zeros((16,), f32)
            mo_cls[sl] = jnp.zeros((16,), f32)
            mo_y1[sl] = jnp.zeros((16,), f32)
            mo_x1[sl] = jnp.zeros((16,), f32)
            mo_y2[sl] = jnp.zeros((16,), f32)
            mo_x2[sl] = jnp.zeros((16,), f32)

        zero16 = jnp.zeros((16,), i32)
        for j in range(8):
            cvec = jnp.full((16,), j * 16, i32) + lane
            if j < 5:
                hv = plsc.load_gather(cs_m, [cvec, zero16])
                csv[pl.ds(j * 16, 16)] = jnp.where(
                    cvec < C, hv, NEG)
            else:
                csv[pl.ds(j * 16, 16)] = jnp.full((16,), NEG, f32)
            civ[pl.ds(j * 16, 16)] = zero16

        def head_amx(j, carry):
            m, cb2 = carry
            v = csv[pl.ds(j * 16, 16)]
            cvec = jnp.full((16,), j * 16, i32) + lane
            upd = v > m
            return jnp.where(upd, v, m), jnp.where(upd, cvec, cb2)

        def pick_head():
            m, cb2 = lax.fori_loop(
                0, 5, head_amx, (jnp.full((16,), NEG, f32), zero16i))
            M = jnp.max(m)
            cstar = jnp.min(jnp.where(m == M, cb2, 1 << 20))
            return M, cstar

        M0, c0m = pick_head()

        def m_cond(carry):
            k, Mc, _ = carry
            return (Mc > 0.0) & (k < _MAX_DET)

        def m_body(carry):
            k, Mc, cst = carry
            kv = jnp.full((16,), k, i32)
            lane0 = lane == 0
            csplat = jnp.full((16,), cst, i32)
            hp = plsc.load_gather(civ, [csplat])
            orig = plsc.load_gather(ci_m, [csplat, hp])
            plsc.store_scatter(mo_conf, [kv], jnp.full((16,), Mc, f32),
                               mask=lane0)
            plsc.store_scatter(mo_cls, [kv], csplat.astype(f32),
                               mask=lane0)
            plsc.store_scatter(mo_y1, [kv], plsc.load_gather(y1_v, [orig]),
                               mask=lane0)
            plsc.store_scatter(mo_x1, [kv], plsc.load_gather(x1_v, [orig]),
                               mask=lane0)
            plsc.store_scatter(mo_y2, [kv], plsc.load_gather(y2_v, [orig]),
                               mask=lane0)
            plsc.store_scatter(mo_x2, [kv], plsc.load_gather(x2_v, [orig]),
                               mask=lane0)
            hp2 = hp + 1
            plsc.store_scatter(civ, [csplat], hp2, mask=lane0)
            newhead = plsc.load_gather(cs_m, [csplat, hp2])
            plsc.store_scatter(csv, [csplat], newhead, mask=lane0)
            M2, c2 = pick_head()
            return k + 1, M2, c2

        kf, _, _ = lax.while_loop(m_cond, m_body,
                                  (jnp.array(0, i32), M0, c0m))
        plsc.store_scatter(mo_num, [zero16i], jnp.full((16,), kf, i32),
                           mask=lane == 0)

        pltpu.async_copy(mo_conf, conf_h.at[bm], sem_m)
        pltpu.async_copy(mo_cls, cls_h.at[bm], sem_m)
        pltpu.async_copy(mo_y1, box_h.at[bm, 0], sem_m)
        pltpu.async_copy(mo_x1, box_h.at[bm, 1], sem_m)
        pltpu.async_copy(mo_y2, box_h.at[bm, 2], sem_m)
        pltpu.async_copy(mo_x2, box_h.at[bm, 3], sem_m)
        pltpu.async_copy(mo_num, num_h.at[bm], sem_m)
        pltpu.make_async_copy(mo_conf, conf_h.at[bm], sem_m).wait()
        pltpu.make_async_copy(mo_cls, cls_h.at[bm], sem_m).wait()
        pltpu.make_async_copy(mo_y1, box_h.at[bm, 0], sem_m).wait()
        pltpu.make_async_copy(mo_x1, box_h.at[bm, 1], sem_m).wait()
        pltpu.make_async_copy(mo_y2, box_h.at[bm, 2], sem_m).wait()
        pltpu.make_async_copy(mo_x2, box_h.at[bm, 3], sem_m).wait()
        pltpu.make_async_copy(mo_num, num_h.at[bm], sem_m).wait()


def _run_sc_stage(scores2d, y1, x1, y2, x2, area):
    BC, NP = scores2d.shape
    B = y1.shape[0]
    sc_nms = functools.partial(
        pl.kernel,
        mesh=plsc.VectorSubcoreMesh(core_axis_name="c", subcore_axis_name="s"),
        compiler_params=pltpu.CompilerParams(needs_layout_passes=False),
        out_type=[
            jax.ShapeDtypeStruct((BC, _T_PAD), f32),
            jax.ShapeDtypeStruct((BC, _T_PAD), i32),
            jax.ShapeDtypeStruct((B, _T_PAD), f32),
            jax.ShapeDtypeStruct((B, _T_PAD), f32),
            jax.ShapeDtypeStruct((B, 4, _T_PAD), f32),
            jax.ShapeDtypeStruct((B, 16), i32),
        ],
        scratch_types=[
            pltpu.VMEM((NP,), f32),
            pltpu.VMEM((NP,), f32),
            pltpu.VMEM((NP,), f32),
            pltpu.VMEM((NP,), f32),
            pltpu.VMEM((NP,), f32),
            pltpu.VMEM((NP,), f32),
            pltpu.VMEM((_NC_PAD,), f32),
            pltpu.VMEM((_NC_PAD,), i32),
            pltpu.VMEM((_NC_PAD,), f32),
            pltpu.VMEM((_NC_PAD,), f32),
            pltpu.VMEM((_NC_PAD,), f32),
            pltpu.VMEM((_NC_PAD,), f32),
            pltpu.VMEM((_NC_PAD,), f32),
            pltpu.VMEM((_T_PAD,), f32),
            pltpu.VMEM((_T_PAD,), i32),
            pltpu.VMEM((80, _T_PAD), f32),
            pltpu.VMEM((80, _T_PAD), i32),
            pltpu.VMEM((_T_PAD,), f32),
            pltpu.VMEM((_T_PAD,), f32),
            pltpu.VMEM((_T_PAD,), f32),
            pltpu.VMEM((_T_PAD,), f32),
            pltpu.VMEM((_T_PAD,), f32),
            pltpu.VMEM((_T_PAD,), f32),
            pltpu.VMEM((16,), i32),
            pltpu.SemaphoreType.DMA,
            pltpu.SemaphoreType.DMA,
            pltpu.SemaphoreType.DMA,
            pltpu.SemaphoreType.DMA,
            pltpu.SemaphoreType.DMA,
        ],
    )(_sc_nms_body)
    return sc_nms(scores2d, y1, x1, y2, x2, area)


def kernel(box_pred, confidence_pred):
    B, N, C = confidence_pred.shape
    NP = _N_PAD
    T = _MAX_DET
    conf_t = jnp.pad(jnp.transpose(confidence_pred, (0, 2, 1)),
                     ((0, 0), (0, 0), (0, NP - N)))
    box_t = jnp.pad(jnp.transpose(box_pred, (0, 2, 1)),
                    ((0, 0), (0, 0), (0, NP - N)))

    scores, area = pl.pallas_call(
        _softmax_body,
        out_shape=[
            jax.ShapeDtypeStruct((B, C, NP), f32),
            jax.ShapeDtypeStruct((B, NP), f32),
        ],
    )(conf_t, box_t)

    _, _, conf_h, cls_h, box_h, num_h = _run_sc_stage(
        scores.reshape(B * C, NP),
        box_t[:, 0, :], box_t[:, 1, :], box_t[:, 2, :], box_t[:, 3, :],
        area)

    boxes_out = jnp.transpose(box_h, (0, 2, 1))[:, :T, :]
    conf_out = conf_h[:, :T]
    cls_out = cls_h[:, :T]
    num = num_h[:, 0]
    return boxes_out, conf_out, cls_out, num


# speedup vs baseline: 2.9508x; 1.0952x over previous
import functools
import jax
import jax.numpy as jnp
from jax import lax
from jax.experimental import pallas as pl
from jax.experimental.pallas import tpu as pltpu
from jax.experimental.pallas import tpu_sc as plsc

_IOU_THR = 0.5
_CONF_THR = 0.05
_MAX_DET = 100
_N_PAD = 1024
_NC_PAD = _N_PAD + 16
_T_PAD = 128
_CHUNK = 256

f32 = jnp.float32
i32 = jnp.int32


def _sc_body(logit_h, y1_h, x1_h, y2_h, x2_h,
             scores_h, cs_h, ci_h, conf_h, cls_h, box_h, num_h,
             zbuf, s_v, y1_v, x1_v, y2_v, x2_v,
             sc_c, idx_c, y1c, x1c, y2c, x2c, areac, csv, civ,
             cs_m, ci_m, mo_conf, mo_cls,
             mo_y1, mo_x1, mo_y2, mo_x2, mo_num,
             sem_b, sem_s, sem_oc, sem_oi, sem_m, sem_w):
    C = 80
    TASKS_PER = C // 4
    NV = _N_PAD // 16
    CV = _CHUNK // 16
    NEG = f32(-jnp.inf)
    lane = lax.iota(i32, 16)
    zero16i = jnp.zeros((16,), i32)

    cid = lax.axis_index("c")
    sid = lax.axis_index("s")
    wid = cid * 16 + sid
    b = wid // 4
    q = wid % 4

    pltpu.async_copy(
        logit_h.at[b, :, pl.ds(q * _CHUNK, _CHUNK)], zbuf, sem_b).wait()

    def mx_body(c, carry):
        return tuple(
            jnp.maximum(carry[v], zbuf[c, pl.ds(v * 16, 16)])
            for v in range(CV))

    mxs = lax.fori_loop(0, C, mx_body,
                        tuple(jnp.full((16,), NEG, f32)
                              for _ in range(CV)))

    def es_body(c, carry):
        outs = []
        for v in range(CV):
            e = jnp.exp(zbuf[c, pl.ds(v * 16, 16)] - mxs[v])
            zbuf[c, pl.ds(v * 16, 16)] = e
            outs.append(carry[v] + e)
        return tuple(outs)

    sums = lax.fori_loop(0, C, es_body,
                         tuple(jnp.zeros((16,), f32) for _ in range(CV)))

    def dv_body(c, _):
        for v in range(CV):
            zbuf[c, pl.ds(v * 16, 16)] = (
                zbuf[c, pl.ds(v * 16, 16)] / sums[v])
        pltpu.async_copy(
            zbuf.at[c],
            scores_h.at[b * C + c, pl.ds(q * _CHUNK, _CHUNK)], sem_w)
        return 0

    lax.fori_loop(0, C, dv_body, 0)

    def dr_body(c, _):
        pltpu.make_async_copy(
            zbuf.at[c],
            scores_h.at[b * C + c, pl.ds(q * _CHUNK, _CHUNK)],
            sem_w).wait()
        return 0

    lax.fori_loop(0, C, dr_body, 0)

    plsc.subcore_barrier()

    c0 = q * TASKS_PER
    cb = [pltpu.async_copy(y1_h.at[b], y1_v, sem_b),
          pltpu.async_copy(x1_h.at[b], x1_v, sem_b),
          pltpu.async_copy(y2_h.at[b], y2_v, sem_b),
          pltpu.async_copy(x2_h.at[b], x2_v, sem_b)]
    for c in cb:
        c.wait()

    def task_body(ti, _):
        t = b * C + c0 + ti
        c_s = pltpu.async_copy(scores_h.at[t], s_v, sem_s)

        @pl.when(ti > 0)
        def _wait_prev_out():
            pltpu.make_async_copy(csv, cs_h.at[t], sem_oc).wait()
            pltpu.make_async_copy(civ, ci_h.at[t], sem_oi).wait()

        for j in range(_T_PAD // 16):
            csv[pl.ds(j * 16, 16)] = jnp.full((16,), -1.0, f32)
            civ[pl.ds(j * 16, 16)] = zero16i
        c_s.wait()

        def compact(i, offv):
            sl = pl.ds(i * 16, 16)
            sv = s_v[sl]
            msk = sv > _CONF_THR
            cum = plsc.cumsum(msk.astype(i32))
            pos = offv + cum - 1
            plsc.store_scatter(sc_c, [pos], sv, mask=msk)
            nidx = jnp.full((16,), i * 16, i32) + lane
            plsc.store_scatter(idx_c, [pos], nidx, mask=msk)
            return offv + plsc.all_reduce_population_count(msk)

        offv = lax.fori_loop(0, NV, compact, zero16i)
        K = jnp.max(offv)
        nv = (K + 15) // 16
        Kv = jnp.full((16,), K, i32)

        def gatherc(j, _):
            sl = pl.ds(j * 16, 16)
            posv = jnp.full((16,), j * 16, i32) + lane
            idxv = jnp.where(posv < Kv, idx_c[sl], 0)
            idx_c[sl] = idxv
            vy1 = plsc.load_gather(y1_v, [idxv])
            vx1 = plsc.load_gather(x1_v, [idxv])
            vy2 = plsc.load_gather(y2_v, [idxv])
            vx2 = plsc.load_gather(x2_v, [idxv])
            y1c[sl] = vy1
            x1c[sl] = vx1
            y2c[sl] = vy2
            x2c[sl] = vx2
            areac[sl] = (jnp.maximum(vy2 - vy1, 0.0) *
                         jnp.maximum(vx2 - vx1, 0.0))
            sc_c[sl] = jnp.where(posv < Kv, sc_c[sl], NEG)
            return 0

        lax.fori_loop(0, nv, gatherc, 0)

        def amx(j, carry):
            m, pb = carry
            v = sc_c[pl.ds(j * 16, 16)]
            posv = jnp.full((16,), j * 16, i32) + lane
            upd = v > m
            return jnp.where(upd, v, m), jnp.where(upd, posv, pb)

        m, pb = lax.fori_loop(0, nv, amx,
                              (jnp.full((16,), NEG, f32), zero16i))
        M = jnp.max(m)
        pstar = jnp.min(jnp.where(m == M, pb, 1 << 20))

        def g_cond(carry):
            tk, Mc, _ = carry
            return (Mc > _CONF_THR) & (tk < _MAX_DET)

        def g_body(carry):
            tk, Mc, p = carry
            tkv = jnp.full((16,), tk, i32)
            lane0 = lane == 0
            psplat = jnp.full((16,), p, i32)
            orig = plsc.load_gather(idx_c, [psplat])
            plsc.store_scatter(csv, [tkv], jnp.full((16,), Mc, f32),
                               mask=lane0)
            plsc.store_scatter(civ, [tkv], orig, mask=lane0)
            sy1 = plsc.load_gather(y1c, [psplat])
            sx1 = plsc.load_gather(x1c, [psplat])
            sy2 = plsc.load_gather(y2c, [psplat])
            sx2 = plsc.load_gather(x2c, [psplat])
            asel = plsc.load_gather(areac, [psplat])

            def sup(j, carry2):
                m2, pb2 = carry2
                sl = pl.ds(j * 16, 16)
                sv = sc_c[sl]
                iy1 = jnp.maximum(sy1, y1c[sl])
                ix1 = jnp.maximum(sx1, x1c[sl])
                iy2 = jnp.minimum(sy2, y2c[sl])
                ix2 = jnp.minimum(sx2, x2c[sl])
                inter = (jnp.maximum(iy2 - iy1, 0.0) *
                         jnp.maximum(ix2 - ix1, 0.0))
                union = asel + areac[sl] - inter
                iou = inter / (union + 1e-8)
                posv = jnp.full((16,), j * 16, i32) + lane
                kill = (iou > _IOU_THR) | (posv == psplat)
                sv2 = jnp.where(kill, NEG, sv)
                sc_c[sl] = sv2
                upd = sv2 > m2
                return jnp.where(upd, sv2, m2), jnp.where(upd, posv, pb2)

            m2, pb2 = lax.fori_loop(0, nv, sup,
                                    (jnp.full((16,), NEG, f32), zero16i))
            M2 = jnp.max(m2)
            p2 = jnp.min(jnp.where(m2 == M2, pb2, 1 << 20))
            return tk + 1, M2, p2

        lax.while_loop(g_cond, g_body, (jnp.array(0, i32), M, pstar))
        pltpu.async_copy(csv, cs_h.at[t], sem_oc)
        pltpu.async_copy(civ, ci_h.at[t], sem_oi)
        return 0

    lax.fori_loop(0, TASKS_PER, task_body, 0)
    t_last = b * C + c0 + TASKS_PER - 1
    pltpu.make_async_copy(csv, cs_h.at[t_last], sem_oc).wait()
    pltpu.make_async_copy(civ, ci_h.at[t_last], sem_oi).wait()

    plsc.subcore_barrier()

    @pl.when(sid < 4)
    def _merge():
        bm = cid * 4 + sid
        cm = [pltpu.async_copy(cs_h.at[pl.ds(bm * C, C)], cs_m, sem_m),
              pltpu.async_copy(ci_h.at[pl.ds(bm * C, C)], ci_m, sem_m),
              pltpu.async_copy(y1_h.at[bm], y1_v, sem_m),
              pltpu.async_copy(x1_h.at[bm], x1_v, sem_m),
              pltpu.async_copy(y2_h.at[bm], y2_v, sem_m),
              pltpu.async_copy(x2_h.at[bm], x2_v, sem_m)]
        for c in cm:
            c.wait()
        for j in range(_T_PAD // 16):
            sl = pl.ds(j * 16, 16)
            mo_conf[sl] = jnp.zeros((16,), f32)
            mo_cls[sl] = jnp.zeros((16,), f32)
            mo_y1[sl] = jnp.zeros((16,), f32)
            mo_x1[sl] = jnp.zeros((16,), f32)
            mo_y2[sl] = jnp.zeros((16,), f32)
            mo_x2[sl] = jnp.zeros((16,), f32)

        zero16 = jnp.zeros((16,), i32)
        for j in range(8):
            cvec = jnp.full((16,), j * 16, i32) + lane
            if j < 5:
                hv = plsc.load_gather(cs_m, [cvec, zero16])
                csv[pl.ds(j * 16, 16)] = hv
            else:
                csv[pl.ds(j * 16, 16)] = jnp.full((16,), NEG, f32)
            civ[pl.ds(j * 16, 16)] = zero16

        def head_amx(j, carry):
            m, cb2 = carry
            v = csv[pl.ds(j * 16, 16)]
            cvec = jnp.full((16,), j * 16, i32) + lane
            upd = v > m
            return jnp.where(upd, v, m), jnp.where(upd, cvec, cb2)

        def pick_head():
            m, cb2 = lax.fori_loop(
                0, 5, head_amx, (jnp.full((16,), NEG, f32), zero16i))
            M = jnp.max(m)
            cstar = jnp.min(jnp.where(m == M, cb2, 1 << 20))
            return M, cstar

        M0, c0m = pick_head()

        def m_cond(carry):
            k, Mc, _ = carry
            return (Mc > 0.0) & (k < _MAX_DET)

        def m_body(carry):
            k, Mc, cst = carry
            kv = jnp.full((16,), k, i32)
            lane0 = lane == 0
            csplat = jnp.full((16,), cst, i32)
            hp = plsc.load_gather(civ, [csplat])
            orig = plsc.load_gather(ci_m, [csplat, hp])
            plsc.store_scatter(mo_conf, [kv], jnp.full((16,), Mc, f32),
                               mask=lane0)
            plsc.store_scatter(mo_cls, [kv], csplat.astype(f32),
                               mask=lane0)
            plsc.store_scatter(mo_y1, [kv], plsc.load_gather(y1_v, [orig]),
                               mask=lane0)
            plsc.store_scatter(mo_x1, [kv], plsc.load_gather(x1_v, [orig]),
                               mask=lane0)
            plsc.store_scatter(mo_y2, [kv], plsc.load_gather(y2_v, [orig]),
                               mask=lane0)
            plsc.store_scatter(mo_x2, [kv], plsc.load_gather(x2_v, [orig]),
                               mask=lane0)
            hp2 = hp + 1
            plsc.store_scatter(civ, [csplat], hp2, mask=lane0)
            newhead = plsc.load_gather(cs_m, [csplat, hp2])
            plsc.store_scatter(csv, [csplat], newhead, mask=lane0)
            M2, c2 = pick_head()
            return k + 1, M2, c2

        kf, _, _ = lax.while_loop(m_cond, m_body,
                                  (jnp.array(0, i32), M0, c0m))
        plsc.store_scatter(mo_num, [zero16i], jnp.full((16,), kf, i32),
                           mask=lane == 0)

        pltpu.async_copy(mo_conf, conf_h.at[bm], sem_m)
        pltpu.async_copy(mo_cls, cls_h.at[bm], sem_m)
        pltpu.async_copy(mo_y1, box_h.at[bm, 0], sem_m)
        pltpu.async_copy(mo_x1, box_h.at[bm, 1], sem_m)
        pltpu.async_copy(mo_y2, box_h.at[bm, 2], sem_m)
        pltpu.async_copy(mo_x2, box_h.at[bm, 3], sem_m)
        pltpu.async_copy(mo_num, num_h.at[bm], sem_m)
        pltpu.make_async_copy(mo_conf, conf_h.at[bm], sem_m).wait()
        pltpu.make_async_copy(mo_cls, cls_h.at[bm], sem_m).wait()
        pltpu.make_async_copy(mo_y1, box_h.at[bm, 0], sem_m).wait()
        pltpu.make_async_copy(mo_x1, box_h.at[bm, 1], sem_m).wait()
        pltpu.make_async_copy(mo_y2, box_h.at[bm, 2], sem_m).wait()
        pltpu.make_async_copy(mo_x2, box_h.at[bm, 3], sem_m).wait()
        pltpu.make_async_copy(mo_num, num_h.at[bm], sem_m).wait()


def _run_sc_stage(logits, y1, x1, y2, x2):
    B, C, NP = logits.shape
    sc = functools.partial(
        pl.kernel,
        mesh=plsc.VectorSubcoreMesh(core_axis_name="c", subcore_axis_name="s"),
        compiler_params=pltpu.CompilerParams(needs_layout_passes=False),
        out_type=[
            jax.ShapeDtypeStruct((B * C, NP), f32),
            jax.ShapeDtypeStruct((B * C, _T_PAD), f32),
            jax.ShapeDtypeStruct((B * C, _T_PAD), i32),
            jax.ShapeDtypeStruct((B, _T_PAD), f32),
            jax.ShapeDtypeStruct((B, _T_PAD), f32),
            jax.ShapeDtypeStruct((B, 4, _T_PAD), f32),
            jax.ShapeDtypeStruct((B, 16), i32),
        ],
        scratch_types=[
            pltpu.VMEM((C, _CHUNK), f32),
            pltpu.VMEM((NP,), f32),
            pltpu.VMEM((NP,), f32),
            pltpu.VMEM((NP,), f32),
            pltpu.VMEM((NP,), f32),
            pltpu.VMEM((NP,), f32),
            pltpu.VMEM((_NC_PAD,), f32),
            pltpu.VMEM((_NC_PAD,), i32),
            pltpu.VMEM((_NC_PAD,), f32),
            pltpu.VMEM((_NC_PAD,), f32),
            pltpu.VMEM((_NC_PAD,), f32),
            pltpu.VMEM((_NC_PAD,), f32),
            pltpu.VMEM((_NC_PAD,), f32),
            pltpu.VMEM((_T_PAD,), f32),
            pltpu.VMEM((_T_PAD,), i32),
            pltpu.VMEM((80, _T_PAD), f32),
            pltpu.VMEM((80, _T_PAD), i32),
            pltpu.VMEM((_T_PAD,), f32),
            pltpu.VMEM((_T_PAD,), f32),
            pltpu.VMEM((_T_PAD,), f32),
            pltpu.VMEM((_T_PAD,), f32),
            pltpu.VMEM((_T_PAD,), f32),
            pltpu.VMEM((_T_PAD,), f32),
            pltpu.VMEM((16,), i32),
            pltpu.SemaphoreType.DMA,
            pltpu.SemaphoreType.DMA,
            pltpu.SemaphoreType.DMA,
            pltpu.SemaphoreType.DMA,
            pltpu.SemaphoreType.DMA,
            pltpu.SemaphoreType.DMA,
        ],
    )(_sc_body)
    return sc(logits, y1, x1, y2, x2)


def kernel(box_pred, confidence_pred):
    B, N, C = confidence_pred.shape
    NP = _N_PAD
    T = _MAX_DET
    conf_t = jnp.pad(jnp.transpose(confidence_pred, (0, 2, 1)),
                     ((0, 0), (0, 0), (0, NP - N)))
    box_t = jnp.pad(jnp.transpose(box_pred, (0, 2, 1)),
                    ((0, 0), (0, 0), (0, NP - N)))

    outs = _run_sc_stage(
        conf_t,
        box_t[:, 0, :], box_t[:, 1, :], box_t[:, 2, :], box_t[:, 3, :])
    _, _, _, conf_h, cls_h, box_h, num_h = outs

    boxes_out = jnp.transpose(box_h, (0, 2, 1))[:, :T, :]
    conf_out = conf_h[:, :T]
    cls_out = cls_h[:, :T]
    num = num_h[:, 0]
    return boxes_out, conf_out, cls_out, num
